# baseline (speedup 1.0000x reference)
import functools

import jax
import jax.numpy as jnp
from jax import lax
from jax.experimental import pallas as pl
from jax.experimental.pallas import tpu as pltpu


def _l1_kernel(x_ref, s_ref, g_ref, b_ref, o_ref):
    x = x_ref[...]
    acc = jnp.zeros((64, 86 * 32), jnp.float32) + b_ref[...]
    for di in range(4):
        t = jnp.dot(s_ref[di], x, preferred_element_type=jnp.float32)
        acc = acc + jnp.dot(t, g_ref[di], preferred_element_type=jnp.float32)
    o_ref[...] = jnp.maximum(acc, 0.0)


def _l1_conv(x_pad, s, g, b):
    bsz = x_pad.shape[0]
    return pl.pallas_call(
        _l1_kernel,
        grid=(bsz,),
        in_specs=[
            pl.BlockSpec((None, 130, 174), lambda i: (i, 0, 0)),
            pl.BlockSpec((4, 64, 130), lambda i: (0, 0, 0)),
            pl.BlockSpec((4, 174, 86 * 32), lambda i: (0, 0, 0)),
            pl.BlockSpec((1, 86 * 32), lambda i: (0, 0)),
        ],
        out_specs=pl.BlockSpec((None, 64, 86 * 32), lambda i: (i, 0, 0)),
        out_shape=jax.ShapeDtypeStruct((bsz, 64, 86 * 32), jnp.float32),
    )(x_pad, s, g, b)


def _l1_weights(enc_w1, enc_b1):
    cols = lax.broadcasted_iota(jnp.int32, (174, 86), 0)
    xs = lax.broadcasted_iota(jnp.int32, (174, 86), 1)
    w = enc_w1[:, 0, :, :]
    g = jnp.zeros((4, 174, 86, 32), jnp.float32)
    for dj in range(4):
        m = (cols == 2 * xs + dj).astype(jnp.float32)
        g = g + m[None, :, :, None] * w[:, :, dj].T[:, None, None, :]
    rows = lax.broadcasted_iota(jnp.int32, (4, 64, 130), 1)
    rsel = lax.broadcasted_iota(jnp.int32, (4, 64, 130), 2)
    dis = lax.broadcasted_iota(jnp.int32, (4, 64, 130), 0)
    s = (rsel == 2 * rows + dis).astype(jnp.float32)
    return s, g.reshape(4, 174, 86 * 32), jnp.tile(enc_b1, (86,))[None, :]


def _conv_s2_kernel(a_ref, w_ref, b_ref, o_ref, *, ho, wo, c):
    taps = []
    for di in range(4):
        for dj in range(4):
            v = a_ref[di:di + 2 * ho - 1:2, dj:dj + 2 * wo - 1:2, :]
            taps.append(v.reshape(ho * wo, c))
    patch = jnp.concatenate(taps, axis=1)
    acc = jnp.dot(patch, w_ref[...], preferred_element_type=jnp.float32)
    o_ref[...] = jnp.maximum(acc + b_ref[...], 0.0)


def _conv_s2(x_pad, w, b, ho, wo):
    bsz, hp, wp, c = x_pad.shape
    n = w.shape[1]
    return pl.pallas_call(
        functools.partial(_conv_s2_kernel, ho=ho, wo=wo, c=c),
        grid=(bsz,),
        in_specs=[
            pl.BlockSpec((None, hp, wp, c), lambda i: (i, 0, 0, 0)),
            pl.BlockSpec((16 * c, n), lambda i: (0, 0)),
            pl.BlockSpec((1, n), lambda i: (0, 0)),
        ],
        out_specs=pl.BlockSpec((None, ho * wo, n), lambda i: (i, 0, 0)),
        out_shape=jax.ShapeDtypeStruct((bsz, ho * wo, n), jnp.float32),
    )(x_pad, w, b)


def _conv_3x3_kernel(a_ref, w_ref, b_ref, o_ref, *, h, wd, c):
    taps = []
    for da in range(3):
        for db in range(3):
            v = a_ref[da:da + h, db:db + wd, :]
            taps.append(v.reshape(h * wd, c))
    patch = jnp.concatenate(taps, axis=1)
    acc = jnp.dot(patch, w_ref[...], preferred_element_type=jnp.float32)
    o_ref[...] = jnp.maximum(acc + b_ref[...], 0.0)


def _conv_3x3(x_pad, w, b, h, wd):
    bsz, hp, wp, c = x_pad.shape
    n = w.shape[1]
    return pl.pallas_call(
        functools.partial(_conv_3x3_kernel, h=h, wd=wd, c=c),
        grid=(bsz,),
        in_specs=[
            pl.BlockSpec((None, hp, wp, c), lambda i: (i, 0, 0, 0)),
            pl.BlockSpec((9 * c, n), lambda i: (0, 0)),
            pl.BlockSpec((1, n), lambda i: (0, 0)),
        ],
        out_specs=pl.BlockSpec((None, h * wd, n), lambda i: (i, 0, 0)),
        out_shape=jax.ShapeDtypeStruct((bsz, h * wd, n), jnp.float32),
    )(x_pad, w, b)


def _vq_kernel(z_ref, emb_ref, zq_ref, codes_ref, ssq_ref):
    z = z_ref[...]
    emb = emb_ref[...]
    score = jnp.sum(z * z, axis=1)[:, None] + jnp.sum(emb * emb, axis=1)[None, :]
    score = score - 2.0 * jnp.dot(z, emb.T, preferred_element_type=jnp.float32)
    codes = jnp.argmin(score, axis=1).astype(jnp.int32)
    onehot = (lax.broadcasted_iota(jnp.int32, score.shape, 1)
              == codes[:, None]).astype(jnp.float32)
    zq = jnp.dot(onehot, emb, preferred_element_type=jnp.float32)
    zq_ref[...] = zq
    codes_ref[...] = codes[:, None]
    diff = zq - z
    part = jnp.sum(diff * diff, keepdims=True)

    @pl.when(pl.program_id(0) == 0)
    def _init():
        ssq_ref[...] = jnp.zeros_like(ssq_ref)

    ssq_ref[...] += part


def _vq(z_flat, emb):
    m = z_flat.shape[0]
    bm = 1344
    zq, codes, ssq = pl.pallas_call(
        _vq_kernel,
        grid=(m // bm,),
        in_specs=[
            pl.BlockSpec((bm, 128), lambda i: (i, 0)),
            pl.BlockSpec((512, 128), lambda i: (0, 0)),
        ],
        out_specs=[
            pl.BlockSpec((bm, 128), lambda i: (i, 0)),
            pl.BlockSpec((bm, 1), lambda i: (i, 0)),
            pl.BlockSpec((1, 1), lambda i: (0, 0)),
        ],
        out_shape=[
            jax.ShapeDtypeStruct((m, 128), jnp.float32),
            jax.ShapeDtypeStruct((m, 1), jnp.int32),
            jax.ShapeDtypeStruct((1, 1), jnp.float32),
        ],
    )(z_flat, emb)
    return zq, codes[:, 0], ssq[0, 0]


def _final_kernel(d_ref, w_ref, b_ref, o_ref):
    w = w_ref[...]
    acc = jnp.zeros((128, 168), jnp.float32) + b_ref[0, 0]
    for c in range(16):
        for dy in range(3):
            for dx in range(3):
                acc = acc + d_ref[c, dy:dy + 128, dx:dx + 168] * w[c, dy, dx]
    o_ref[...] = 1.0 / (1.0 + jnp.exp(-acc))


def _final_conv(d_nchw_pad, w, b):
    bsz = d_nchw_pad.shape[0]
    return pl.pallas_call(
        _final_kernel,
        grid=(bsz,),
        in_specs=[
            pl.BlockSpec((None, 16, 130, 170), lambda i: (i, 0, 0, 0)),
            pl.BlockSpec((16, 3, 3), lambda i: (0, 0, 0)),
            pl.BlockSpec((1, 1), lambda i: (0, 0)),
        ],
        out_specs=pl.BlockSpec((None, 128, 168), lambda i: (i, 0, 0)),
        out_shape=jax.ShapeDtypeStruct((bsz, 128, 168), jnp.float32),
    )(d_nchw_pad, w, b)


def _pad_hw(x):
    return jnp.pad(x, ((0, 0), (1, 1), (1, 1), (0, 0)))


def _enc_weight(w):
    return w.transpose(2, 3, 1, 0).reshape(-1, w.shape[0])


def _dec_weight9(w):
    ci, co = w.shape[0], w.shape[1]
    w9 = jnp.zeros((3, 3, ci, 2, 2, co), jnp.float32)
    ytaps = [(0, 0, 3), (0, 1, 1), (1, 1, 2), (1, 2, 0)]
    for ey, a, ky in ytaps:
        for ex, b, kx in ytaps:
            w9 = w9.at[a, b, :, ey, ex, :].set(w[:, :, ky, kx])
    return w9.reshape(9 * ci, 4 * co)


def _dec_layer(x_nhwc, w, bias, h, wd):
    bsz, _, _, ci = x_nhwc.shape
    co = w.shape[1]
    out = _conv_3x3(_pad_hw(x_nhwc), _dec_weight9(w),
                    jnp.tile(bias, (1, 4)), h, wd)
    out = out.reshape(bsz, h, wd, 2, 2, co).transpose(0, 1, 3, 2, 4, 5)
    return out.reshape(bsz, 2 * h, 2 * wd, co)


def kernel(x, enc_w1, enc_b1, enc_w2, enc_b2, enc_w3, enc_b3, emb,
           dec_w1, dec_b1, dec_w2, dec_b2, dec_w3, dec_b3, dec_w4, dec_b4):
    beta = 0.25
    bsz = x.shape[0]

    x_pad = jnp.pad(x[:, 0], ((0, 0), (1, 1), (1, 1)))
    s1, g1, b1 = _l1_weights(enc_w1, enc_b1)
    h1 = _l1_conv(x_pad, s1, g1, b1).reshape(bsz, 64, 86, 32)

    h2 = _conv_s2(_pad_hw(h1), _enc_weight(enc_w2), enc_b2[None, :], 32, 43)
    h2 = h2.reshape(bsz, 32, 43, 64)
    z3 = _conv_s2(_pad_hw(h2), _enc_weight(enc_w3), enc_b3[None, :], 16, 21)
    z_flat = z3.reshape(bsz * 16 * 21, 128)

    zq_flat, codes, ssq = _vq(z_flat, emb)
    vq_loss = (1.0 + beta) * ssq / jnp.float32(z_flat.size)

    zq = zq_flat.reshape(bsz, 16, 21, 128)
    d1 = _dec_layer(zq, dec_w1, dec_b1[None, :], 16, 21)
    d2 = _dec_layer(d1, dec_w2, dec_b2[None, :], 32, 42)
    d3 = _dec_layer(d2, dec_w3, dec_b3[None, :], 64, 84)

    d3p = jnp.pad(d3.transpose(0, 3, 1, 2),
                  ((0, 0), (0, 0), (1, 1), (1, 1)))
    xh = _final_conv(d3p, dec_w4[0], dec_b4[None, :])
    x_hat = xh[:, None, :, :]

    return (x_hat, vq_loss, codes.reshape(bsz, 16, 21))
